# baseline (speedup 1.0000x reference)
import jax
import jax.numpy as jnp
from jax.experimental import pallas as pl

TOPK = 8


def _prompt_norm_kernel(pk_ref, pn_ref):
    pk = pk_ref[...]
    ss = jnp.sum(pk * pk, axis=1, keepdims=True)
    pn_ref[...] = pk * jax.lax.rsqrt(jnp.maximum(ss, jnp.float32(1e-12)))


def _main_kernel(x_ref, pn_ref, pr_ref, out_ref, sim_ref, idx_ref, xn_ref,
                 bkn_ref, rs_ref):
    b = pl.program_id(0)
    nb = pl.num_programs(0)
    n = x_ref.shape[1]
    p = pn_ref.shape[0]
    length = pr_ref.shape[1]

    x = x_ref[0]
    xm = jnp.sum(x, axis=0, keepdims=True) / jnp.float32(n)
    ss = jnp.sum(xm * xm, axis=1, keepdims=True)
    xn = xm * jax.lax.rsqrt(jnp.maximum(ss, jnp.float32(1e-12)))
    xn_ref[0] = xn

    sim = jax.lax.dot_general(
        xn, pn_ref[...], (((1,), (1,)), ((), ())),
        precision=jax.lax.Precision.DEFAULT,
        preferred_element_type=jnp.float32)
    sim_ref[0] = sim

    iota = jax.lax.broadcasted_iota(jnp.int32, (1, p), 1)
    kiota = jax.lax.broadcasted_iota(jnp.int32, (1, TOPK), 1)
    vals = sim
    idx_vec = jnp.zeros((1, TOPK), jnp.int32)
    ssum = jnp.float32(0.0)
    for i in range(TOPK):
        m = jnp.max(vals)
        im = jnp.min(jnp.where(vals == m, iota, jnp.int32(p)))
        idx_vec = jnp.where(kiota == i, im, idx_vec)
        ssum = ssum + m
        vals = jnp.where(iota == im, -jnp.inf, vals)
        out_ref[0, pl.ds(i * length, length), :] = pr_ref[im]
        bkn_ref[0, i, :] = pn_ref[im]
    idx_ref[0] = idx_vec

    out_ref[0, pl.ds(TOPK * length, n), :] = x

    @pl.when(b == 0)
    def _():
        rs_ref[...] = jnp.zeros_like(rs_ref)

    rs_ref[...] += ssum / jnp.float32(nb)


def kernel(x_embed, prompt_key, prompt):
    b, n, c = x_embed.shape
    p = prompt_key.shape[0]
    length = prompt.shape[1]
    out_rows = TOPK * length + n

    prompt_norm = pl.pallas_call(
        _prompt_norm_kernel,
        out_shape=jax.ShapeDtypeStruct((p, c), jnp.float32),
    )(prompt_key)

    out_shapes = (
        jax.ShapeDtypeStruct((b, out_rows, c), jnp.float32),
        jax.ShapeDtypeStruct((b, 1, p), jnp.float32),
        jax.ShapeDtypeStruct((b, 1, TOPK), jnp.int32),
        jax.ShapeDtypeStruct((b, 1, c), jnp.float32),
        jax.ShapeDtypeStruct((b, TOPK, c), jnp.float32),
        jax.ShapeDtypeStruct((1, 1), jnp.float32),
    )
    out_specs = (
        pl.BlockSpec((1, out_rows, c), lambda i: (i, 0, 0)),
        pl.BlockSpec((1, 1, p), lambda i: (i, 0, 0)),
        pl.BlockSpec((1, 1, TOPK), lambda i: (i, 0, 0)),
        pl.BlockSpec((1, 1, c), lambda i: (i, 0, 0)),
        pl.BlockSpec((1, TOPK, c), lambda i: (i, 0, 0)),
        pl.BlockSpec((1, 1), lambda i: (0, 0)),
    )
    in_specs = [
        pl.BlockSpec((1, n, c), lambda i: (i, 0, 0)),
        pl.BlockSpec((p, c), lambda i: (0, 0)),
        pl.BlockSpec((p, length, c), lambda i: (0, 0, 0)),
    ]

    prompted, sim, idx, xn, bkn, rs = pl.pallas_call(
        _main_kernel,
        grid=(b,),
        in_specs=in_specs,
        out_specs=out_specs,
        out_shape=out_shapes,
    )(x_embed, prompt_norm, prompt)

    return (prompted,
            sim.reshape(b, p),
            rs.reshape(()),
            idx.reshape(b, TOPK),
            prompt_norm,
            xn.reshape(b, c),
            bkn)


# speedup vs baseline: 1.0556x; 1.0556x over previous
import jax
import jax.numpy as jnp
from jax import lax
from jax.experimental import pallas as pl
from jax.experimental.pallas import tpu as pltpu
from jax.experimental.pallas import tpu_sc as plsc

TOPK = 8
NUM_WORKERS = 32


def _stats_kernel(x_ref, pk_ref, sim_ref, idx_ref, xn_ref, pn_ref, rs_ref,
                  means_ref):
    i = pl.program_id(0)
    rows = x_ref.shape[0]
    n = x_ref.shape[1]
    p = pk_ref.shape[0]
    b = means_ref.shape[0]

    x = x_ref[...]
    means_ref[pl.ds(i * rows, rows), :] = jnp.sum(x, axis=1) / jnp.float32(n)

    @pl.when(i == pl.num_programs(0) - 1)
    def _tail():
        pk = pk_ref[...]
        pss = jnp.sum(pk * pk, axis=1, keepdims=True)
        pn = pk * lax.rsqrt(jnp.maximum(pss, jnp.float32(1e-12)))
        pn_ref[...] = pn

        mm = means_ref[...]
        mss = jnp.sum(mm * mm, axis=1, keepdims=True)
        xn = mm * lax.rsqrt(jnp.maximum(mss, jnp.float32(1e-12)))
        xn_ref[...] = xn

        sim = lax.dot_general(
            xn, pn, (((1,), (1,)), ((), ())),
            precision=lax.Precision.DEFAULT,
            preferred_element_type=jnp.float32)
        sim_ref[...] = sim

        iota = lax.broadcasted_iota(jnp.int32, (b, p), 1)
        kiota = lax.broadcasted_iota(jnp.int32, (b, TOPK), 1)
        vals = sim
        idx_acc = jnp.zeros((b, TOPK), jnp.int32)
        ssum = jnp.float32(0.0)
        for k in range(TOPK):
            m = jnp.max(vals, axis=1, keepdims=True)
            im = jnp.min(jnp.where(vals == m, iota, jnp.int32(p)),
                         axis=1, keepdims=True)
            idx_acc = jnp.where(kiota == k, im, idx_acc)
            ssum = ssum + jnp.sum(m)
            vals = jnp.where(iota == im, -jnp.inf, vals)
        idx_ref[...] = idx_acc
        rs_ref[...] = jnp.full((1, 1), ssum / jnp.float32(b), jnp.float32)


def _make_assemble(b, n, c, p, length):
    d = length * c
    xe = n * c
    orow = TOPK * d + xe
    bpw = b // NUM_WORKERS
    cb = 4 * d

    mesh = plsc.VectorSubcoreMesh(core_axis_name="c", subcore_axis_name="s",
                                  num_cores=2, num_subcores=16)

    def body(x_hbm, prompt_hbm, pn_hbm, idx_hbm, out_hbm, bkn_hbm):
        wid = lax.axis_index("s") * 2 + lax.axis_index("c")
        base = wid * bpw

        def gather_phase(gbuf, bknbuf, idxv, gsem, wsem):
            for j in range(bpw):
                bb = base + j
                pltpu.sync_copy(idx_hbm.at[pl.ds(bb * TOPK, TOPK)], idxv)
                pltpu.async_copy(prompt_hbm.at[idxv], gbuf, gsem).wait()
                pltpu.async_copy(pn_hbm.at[idxv], bknbuf, gsem).wait()
                waits = [
                    pltpu.async_copy(
                        gbuf.at[r],
                        out_hbm.at[pl.ds(bb * orow + r * d, d)], wsem)
                    for r in range(TOPK)
                ]
                waits.append(pltpu.async_copy(bknbuf, bkn_hbm.at[bb], wsem))
                for w in waits:
                    w.wait()

        def copy_phase(cbuf):
            for j in range(bpw):
                bb = base + j
                off = 0
                while off < xe:
                    sz = min(cb, xe - off)
                    pltpu.sync_copy(x_hbm.at[pl.ds(bb * xe + off, sz)],
                                    cbuf.at[pl.ds(0, sz)])
                    pltpu.sync_copy(
                        cbuf.at[pl.ds(0, sz)],
                        out_hbm.at[pl.ds(bb * orow + TOPK * d + off, sz)])
                    off += sz

        pl.run_scoped(gather_phase,
                      pltpu.VMEM((TOPK, d), jnp.float32),
                      pltpu.VMEM((TOPK, c), jnp.float32),
                      pltpu.VMEM((TOPK,), jnp.int32),
                      pltpu.SemaphoreType.DMA,
                      pltpu.SemaphoreType.DMA)
        pl.run_scoped(copy_phase, pltpu.VMEM((cb,), jnp.float32))

    return pl.kernel(
        body,
        out_type=(
            jax.ShapeDtypeStruct((b * orow,), jnp.float32),
            jax.ShapeDtypeStruct((b, TOPK, c), jnp.float32),
        ),
        mesh=mesh,
    )


def kernel(x_embed, prompt_key, prompt):
    b, n, c = x_embed.shape
    p = prompt_key.shape[0]
    length = prompt.shape[1]
    out_rows = TOPK * length + n
    chunk = b // 8

    in_specs = [
        pl.BlockSpec((chunk, n, c), lambda i: (i, 0, 0)),
        pl.BlockSpec((p, c), lambda i: (0, 0)),
    ]
    out_shapes = (
        jax.ShapeDtypeStruct((b, p), jnp.float32),
        jax.ShapeDtypeStruct((b, TOPK), jnp.int32),
        jax.ShapeDtypeStruct((b, c), jnp.float32),
        jax.ShapeDtypeStruct((p, c), jnp.float32),
        jax.ShapeDtypeStruct((1, 1), jnp.float32),
    )
    out_specs = (
        pl.BlockSpec((b, p), lambda i: (0, 0)),
        pl.BlockSpec((b, TOPK), lambda i: (0, 0)),
        pl.BlockSpec((b, c), lambda i: (0, 0)),
        pl.BlockSpec((p, c), lambda i: (0, 0)),
        pl.BlockSpec((1, 1), lambda i: (0, 0)),
    )
    sim, idx, xn, pn, rs = pl.pallas_call(
        _stats_kernel,
        grid=(b // chunk,),
        in_specs=in_specs,
        out_specs=out_specs,
        out_shape=out_shapes,
        scratch_shapes=[pltpu.VMEM((b, c), jnp.float32)],
    )(x_embed, prompt_key)

    assemble = _make_assemble(b, n, c, p, length)
    out_flat, bkn = assemble(
        x_embed.reshape(-1),
        prompt.reshape(p, length * c),
        pn,
        idx.reshape(-1),
    )

    return (out_flat.reshape(b, out_rows, c),
            sim,
            rs.reshape(()),
            idx,
            pn,
            xn,
            bkn)
